# baseline (speedup 1.0000x reference)
import functools

import jax
import jax.numpy as jnp
import numpy as np
from jax import lax
from jax.experimental import pallas as pl
from jax.experimental.pallas import tpu as pltpu
from jax.experimental.pallas import tpu_sc as plsc

_N_SEG = 100
_N_SPAT = 50
_N_OUT = 100
_B = 128
_SEG_W = 112
_SPAT_W = 64
_N_GROUPS = 7
_N_CHAINS = 8
_ROWS = 4
_U = 4


def _build_tables():
    members = [[] for _ in range(_N_OUT)]
    for a in range(_N_SEG):
        for b in range(_N_SPAT):
            s = (a * b) // 10
            if s < _N_OUT:
                members[s].append((a, b))
    counts = [len(m) for m in members]

    b0 = members[0]
    b0_slots = (len(b0) + 15) // 16
    b0_pad = b0 + [(_N_SEG, _N_SPAT)] * (b0_slots * 16 - len(b0))

    rem = sorted(range(1, _N_OUT), key=lambda s: -counts[s])
    group_buckets = [rem[g * 14:(g + 1) * 14] for g in range(6)] + [rem[84:99]]
    later_pool = list(rem[84:99])
    perms, groups = [], []
    res0_lane = None
    pool_i = 0
    for g in range(_N_GROUPS):
        lanes = list(group_buckets[g])
        perm = list(lanes)
        while len(perm) < 16:
            if g == _N_GROUPS - 1 and res0_lane is None:
                res0_lane = len(perm)
                perm.append(0)
            else:
                perm.append(later_pool[pool_i % len(later_pool)])
                pool_i += 1
            lanes.append(None)
        groups.append(lanes)
        perms.append(perm)
    gslots = [max(counts[s] for s in gg if s is not None) for gg in groups]

    def pad_u(n):
        return ((n + _U - 1) // _U) * _U

    chain_lens = tuple([pad_u(b0_slots)] + [pad_u(x) for x in gslots])

    chains = [[[b0_pad[m * 16 + l] if m < b0_slots else (_N_SEG, _N_SPAT)
                for l in range(16)] for m in range(chain_lens[0])]]
    for g in range(_N_GROUPS):
        rows = []
        for m in range(chain_lens[1 + g]):
            row = []
            for l in range(16):
                s = groups[g][l]
                if s is not None and m < counts[s]:
                    row.append(members[s][m])
                else:
                    row.append((_N_SEG, _N_SPAT))
            rows.append(row)
        chains.append(rows)
    pack = np.array([[p[0] + (p[1] << 7) for p in row]
                     for c in chains for row in c], np.int32)
    chain_off = np.cumsum([0] + list(chain_lens))
    params = np.zeros((2, 16), np.int32)
    params[0, :_N_CHAINS] = chain_off[:-1] * 16
    params[1, :_N_CHAINS] = np.array(chain_lens) // _U
    return res0_lane, pack, np.array(perms, np.int32), params


_RES0_LANE, _PACK_TAB, _PERM_TAB, _PARAM_TAB = _build_tables()
_N_SLOT_ROWS = _PACK_TAB.shape[0]


def _sc_body(seg_hbm, spat_hbm, pack_hbm, perm_hbm, param_hbm, out_hbm,
             seg_raw, spat_raw, seg_v, spat_v, pack_v, perm_v, param_v, acc_v,
             out_v):
    wid = lax.axis_index("s") * 2 + lax.axis_index("c")
    base = wid * _ROWS
    pltpu.sync_copy(seg_hbm.at[pl.ds(base, _ROWS)], seg_raw)
    pltpu.sync_copy(spat_hbm.at[pl.ds(base, _ROWS)], spat_raw)
    pltpu.sync_copy(pack_hbm, pack_v)
    pltpu.sync_copy(perm_hbm, perm_v)
    pltpu.sync_copy(param_hbm, param_v)

    one = jnp.float32(1.0)
    zero16 = jnp.zeros((16,), jnp.float32)
    iota = lax.iota(jnp.int32, 16)
    offs_v = param_v[pl.ds(0, 16)]
    trips_v = param_v[pl.ds(16, 16)]

    def stage_body(r, carry):
        row_splat = jnp.zeros((16,), jnp.int32) + r
        for k in range(_SEG_W // 16):
            src = iota + (16 * k)
            g = plsc.load_gather(seg_raw, [row_splat, jnp.minimum(src, _N_SEG - 1)])
            seg_v[pl.ds(r * _SEG_W + 16 * k, 16)] = jnp.where(src < _N_SEG, g, zero16)
        for k in range(_SPAT_W // 16):
            src = iota + (16 * k)
            g = plsc.load_gather(spat_raw, [row_splat, jnp.minimum(src, _N_SPAT - 1)])
            spat_v[pl.ds(r * _SPAT_W + 16 * k, 16)] = jnp.where(src < _N_SPAT, g, zero16)
        return carry

    lax.fori_loop(0, _ROWS, stage_body, 0)

    def row_body(r, carry):
        seg_off = r * _SEG_W
        spat_off = r * _SPAT_W

        def chain_body(c, carry2):
            cbase = jnp.max(jnp.where(iota == c, offs_v, 0))
            trips = jnp.max(jnp.where(iota == c, trips_v, 0))

            def mbody(m, t):
                t0, t1, t2 = t
                moff = cbase + m * (_U * 16)
                for u in range(_U):
                    pk = pack_v[pl.ds(moff + u * 16, 16)]
                    c1 = (pk & 127) + seg_off
                    c2 = lax.shift_right_logical(pk, 7) + spat_off
                    v = (plsc.load_gather(seg_v, [c1])
                         * plsc.load_gather(spat_v, [c2]))
                    a0 = jnp.maximum(t0, v)
                    b0 = jnp.minimum(t0, v)
                    a1 = jnp.maximum(t1, b0)
                    b1 = jnp.minimum(t1, b0)
                    t0, t1, t2 = a0, a1, jnp.maximum(t2, b1)
                return (t0, t1, t2)

            t0, t1, t2 = lax.fori_loop(0, trips, mbody, (zero16, zero16, zero16))
            cs = c * 48
            acc_v[pl.ds(cs, 16)] = t0
            acc_v[pl.ds(cs + 16, 16)] = t1
            acc_v[pl.ds(cs + 32, 16)] = t2
            return carry2

        lax.fori_loop(0, _N_CHAINS, chain_body, 0)

        u0 = acc_v[pl.ds(0, 16)]
        u1 = acc_v[pl.ds(16, 16)]
        u2 = acc_v[pl.ds(32, 16)]
        acc = one
        for _ in range(3):
            mx = jnp.max(u0)
            acc = acc * (one - mx)
            msk = iota == plsc.all_reduce_ffs(u0 == mx)
            u0 = jnp.where(msk, u1, u0)
            u1 = jnp.where(msk, u2, u1)
            u2 = jnp.where(msk, zero16, u2)
        res0 = one - acc

        row_splat = jnp.zeros((16,), jnp.int32) + r

        def merge_body(g, carry3):
            gs = g * 48 + 48
            t0 = acc_v[pl.ds(gs, 16)]
            t1 = acc_v[pl.ds(gs + 16, 16)]
            t2 = acc_v[pl.ds(gs + 32, 16)]
            res = one - (one - t0) * (one - t1) * (one - t2)
            msk = (iota == _RES0_LANE) & (g == _N_GROUPS - 1)
            res = jnp.where(msk, res0, res)
            pos = perm_v[pl.ds(g * 16, 16)]
            plsc.store_scatter(out_v, [row_splat, pos], res)
            return carry3

        lax.fori_loop(0, _N_GROUPS, merge_body, 0)
        return carry

    lax.fori_loop(0, _ROWS, row_body, 0)
    pltpu.sync_copy(out_v, out_hbm.at[pl.ds(base, _ROWS)])


@jax.jit
def _run(seg, spat, pack_flat, perm_flat, param_flat):
    mesh = plsc.VectorSubcoreMesh(core_axis_name="c", subcore_axis_name="s")
    fn = functools.partial(
        pl.kernel,
        mesh=mesh,
        out_type=jax.ShapeDtypeStruct((_B, _N_OUT), jnp.float32),
        compiler_params=pltpu.CompilerParams(needs_layout_passes=False),
        scratch_types=[
            pltpu.VMEM((_ROWS, _N_SEG), jnp.float32),
            pltpu.VMEM((_ROWS, _N_SPAT), jnp.float32),
            pltpu.VMEM((_ROWS * _SEG_W,), jnp.float32),
            pltpu.VMEM((_ROWS * _SPAT_W,), jnp.float32),
            pltpu.VMEM((_N_SLOT_ROWS * 16,), jnp.int32),
            pltpu.VMEM((_N_GROUPS * 16,), jnp.int32),
            pltpu.VMEM((2 * 16,), jnp.int32),
            pltpu.VMEM((_N_CHAINS * 48,), jnp.float32),
            pltpu.VMEM((_ROWS, _N_OUT), jnp.float32),
        ],
    )(_sc_body)
    return fn(seg, spat, pack_flat, perm_flat, param_flat)


def kernel(seg_confidences, spatial_features, rule_weights):
    del rule_weights
    return _run(seg_confidences, spatial_features,
                jnp.asarray(_PACK_TAB).reshape(-1),
                jnp.asarray(_PERM_TAB).reshape(-1),
                jnp.asarray(_PARAM_TAB).reshape(-1))


# speedup vs baseline: 1.0822x; 1.0822x over previous
import functools

import jax
import jax.numpy as jnp
import numpy as np
from jax import lax
from jax.experimental import pallas as pl
from jax.experimental.pallas import tpu as pltpu
from jax.experimental.pallas import tpu_sc as plsc

_N_SEG = 100
_N_SPAT = 50
_N_OUT = 100
_B = 128
_SEG_W = 112
_SPAT_W = 64
_N_GROUPS = 7
_ROWS = 4
_U = 4


def _build_tables():
    members = [[] for _ in range(_N_OUT)]
    for a in range(_N_SEG):
        for b in range(_N_SPAT):
            s = (a * b) // 10
            if s < _N_OUT:
                members[s].append((a, b))
    counts = [len(m) for m in members]

    b0 = members[0]
    b0_slots = (len(b0) + 15) // 16
    b0_pad = b0 + [(_N_SEG, _N_SPAT)] * (b0_slots * 16 - len(b0))

    rem = sorted(range(1, _N_OUT), key=lambda s: -counts[s])
    group_buckets = [rem[g * 14:(g + 1) * 14] for g in range(6)] + [rem[84:99]]
    later_pool = list(rem[84:99])
    perms, groups = [], []
    res0_lane = None
    pool_i = 0
    for g in range(_N_GROUPS):
        lanes = list(group_buckets[g])
        perm = list(lanes)
        while len(perm) < 16:
            if g == _N_GROUPS - 1 and res0_lane is None:
                res0_lane = len(perm)
                perm.append(0)
            else:
                perm.append(later_pool[pool_i % len(later_pool)])
                pool_i += 1
            lanes.append(None)
        groups.append(lanes)
        perms.append(perm)
    gslots = [max(counts[s] for s in gg if s is not None) for gg in groups]

    def pad_u(n):
        return ((n + _U - 1) // _U) * _U

    chain_lens = tuple([pad_u(b0_slots)] + [pad_u(x) for x in gslots])

    chains = [[[b0_pad[m * 16 + l] if m < b0_slots else (_N_SEG, _N_SPAT)
                for l in range(16)] for m in range(chain_lens[0])]]
    for g in range(_N_GROUPS):
        rows = []
        for m in range(chain_lens[1 + g]):
            row = []
            for l in range(16):
                s = groups[g][l]
                if s is not None and m < counts[s]:
                    row.append(members[s][m])
                else:
                    row.append((_N_SEG, _N_SPAT))
            rows.append(row)
        chains.append(rows)
    pack = np.array([[p[0] + (p[1] << 7) for p in row]
                     for c in chains for row in c], np.int32)
    return chain_lens, res0_lane, pack, np.array(perms, np.int32)


_CHAIN_LENS, _RES0_LANE, _PACK_TAB, _PERM_TAB = _build_tables()
_CHAIN_OFF = tuple(int(x) for x in np.cumsum((0,) + _CHAIN_LENS))
_N_SLOTS = int(sum(_CHAIN_LENS))


def _sc_body(seg_hbm, spat_hbm, pack_hbm, perm_hbm, out_hbm,
             seg_raw, spat_raw, seg_v, spat_v, pack_v, perm_v, out_v):
    wid = lax.axis_index("s") * 2 + lax.axis_index("c")
    base = wid * _ROWS
    pltpu.sync_copy(seg_hbm.at[pl.ds(base, _ROWS)], seg_raw)
    pltpu.sync_copy(spat_hbm.at[pl.ds(base, _ROWS)], spat_raw)
    pltpu.sync_copy(pack_hbm, pack_v)
    pltpu.sync_copy(perm_hbm, perm_v)

    one = jnp.float32(1.0)
    zero16 = jnp.zeros((16,), jnp.float32)
    iota = lax.iota(jnp.int32, 16)

    for r in range(_ROWS):
        row_splat = jnp.zeros((16,), jnp.int32) + r
        for k in range(_SEG_W // 16):
            src = iota + (16 * k)
            g = plsc.load_gather(seg_raw, [row_splat, jnp.minimum(src, _N_SEG - 1)])
            seg_v[pl.ds(r * _SEG_W + 16 * k, 16)] = jnp.where(src < _N_SEG, g, zero16)
        for k in range(_SPAT_W // 16):
            src = iota + (16 * k)
            g = plsc.load_gather(spat_raw, [row_splat, jnp.minimum(src, _N_SPAT - 1)])
            spat_v[pl.ds(r * _SPAT_W + 16 * k, 16)] = jnp.where(src < _N_SPAT, g, zero16)

    def row_body(r, carry):
        seg_off = r * _SEG_W
        spat_off = r * _SPAT_W

        def make_chain(c):
            cbase = _CHAIN_OFF[c] * 16

            def mbody(m, t):
                t0, t1, t2 = t
                moff = m * (_U * 16)
                vs = []
                for u in range(_U):
                    pk = pack_v[pl.ds(moff + (cbase + u * 16), 16)]
                    c1 = (pk & 127) + seg_off
                    c2 = lax.shift_right_logical(pk, 7) + spat_off
                    vs.append(plsc.load_gather(seg_v, [c1])
                              * plsc.load_gather(spat_v, [c2]))
                m01 = jnp.maximum(vs[0], vs[1])
                n01 = jnp.minimum(vs[0], vs[1])
                m23 = jnp.maximum(vs[2], vs[3])
                n23 = jnp.minimum(vs[2], vs[3])
                s0 = jnp.maximum(m01, m23)
                x = jnp.minimum(m01, m23)
                y = jnp.maximum(n01, n23)
                s1 = jnp.maximum(x, y)
                s2 = jnp.minimum(x, y)
                c0 = jnp.maximum(t0, s0)
                c1_ = jnp.maximum(jnp.minimum(t0, s0), jnp.maximum(t1, s1))
                c2_ = jnp.maximum(jnp.maximum(t2, s2),
                                  jnp.maximum(jnp.minimum(t1, s0),
                                              jnp.minimum(t0, s1)))
                return (c0, c1_, c2_)

            return lax.fori_loop(0, _CHAIN_LENS[c] // _U, mbody,
                                 (zero16, zero16, zero16))

        ts = [make_chain(c) for c in range(1 + _N_GROUPS)]

        u0, u1, u2 = ts[0]
        acc = one
        for _ in range(3):
            mx = jnp.max(u0)
            acc = acc * (one - mx)
            msk = iota == plsc.all_reduce_ffs(u0 == mx)
            u0 = jnp.where(msk, u1, u0)
            u1 = jnp.where(msk, u2, u1)
            u2 = jnp.where(msk, zero16, u2)
        res0 = one - acc

        row_splat = jnp.zeros((16,), jnp.int32) + r
        for g in range(_N_GROUPS):
            t0, t1, t2 = ts[1 + g]
            res = one - (one - t0) * (one - t1) * (one - t2)
            if g == _N_GROUPS - 1:
                res = jnp.where(iota == _RES0_LANE, res0, res)
            pos = perm_v[pl.ds(g * 16, 16)]
            plsc.store_scatter(out_v, [row_splat, pos], res)
        return carry

    lax.fori_loop(0, _ROWS, row_body, 0)
    pltpu.sync_copy(out_v, out_hbm.at[pl.ds(base, _ROWS)])


@jax.jit
def _run(seg, spat, pack_flat, perm_flat):
    mesh = plsc.VectorSubcoreMesh(core_axis_name="c", subcore_axis_name="s")
    fn = functools.partial(
        pl.kernel,
        mesh=mesh,
        out_type=jax.ShapeDtypeStruct((_B, _N_OUT), jnp.float32),
        compiler_params=pltpu.CompilerParams(
            needs_layout_passes=False,
            skip_device_barrier=True,
        ),
        scratch_types=[
            pltpu.VMEM((_ROWS, _N_SEG), jnp.float32),
            pltpu.VMEM((_ROWS, _N_SPAT), jnp.float32),
            pltpu.VMEM((_ROWS * _SEG_W,), jnp.float32),
            pltpu.VMEM((_ROWS * _SPAT_W,), jnp.float32),
            pltpu.VMEM((_N_SLOTS * 16,), jnp.int32),
            pltpu.VMEM((_N_GROUPS * 16,), jnp.int32),
            pltpu.VMEM((_ROWS, _N_OUT), jnp.float32),
        ],
    )(_sc_body)
    return fn(seg, spat, pack_flat, perm_flat)


def kernel(seg_confidences, spatial_features, rule_weights):
    del rule_weights
    return _run(seg_confidences, spatial_features,
                jnp.asarray(_PACK_TAB).reshape(-1),
                jnp.asarray(_PERM_TAB).reshape(-1))


# speedup vs baseline: 1.0823x; 1.0001x over previous
import functools

import jax
import jax.numpy as jnp
import numpy as np
from jax import lax
from jax.experimental import pallas as pl
from jax.experimental.pallas import tpu as pltpu
from jax.experimental.pallas import tpu_sc as plsc

_N_SEG = 100
_N_SPAT = 50
_N_OUT = 100
_B = 128
_SEG_W = 112
_SPAT_W = 64
_N_GROUPS = 7
_ROWS = 4
_U = 4


def _build_tables():
    members = [[] for _ in range(_N_OUT)]
    for a in range(_N_SEG):
        for b in range(_N_SPAT):
            s = (a * b) // 10
            if s < _N_OUT:
                members[s].append((a, b))
    counts = [len(m) for m in members]

    b0 = members[0]
    b0_slots = (len(b0) + 15) // 16
    b0_pad = b0 + [(_N_SEG, _N_SPAT)] * (b0_slots * 16 - len(b0))

    rem = sorted(range(1, _N_OUT), key=lambda s: -counts[s])
    group_buckets = [rem[g * 14:(g + 1) * 14] for g in range(6)] + [rem[84:99]]
    later_pool = list(rem[84:99])
    perms, groups = [], []
    res0_lane = None
    pool_i = 0
    for g in range(_N_GROUPS):
        lanes = list(group_buckets[g])
        perm = list(lanes)
        while len(perm) < 16:
            if g == _N_GROUPS - 1 and res0_lane is None:
                res0_lane = len(perm)
                perm.append(0)
            else:
                perm.append(later_pool[pool_i % len(later_pool)])
                pool_i += 1
            lanes.append(None)
        groups.append(lanes)
        perms.append(perm)
    gslots = [max(counts[s] for s in gg if s is not None) for gg in groups]

    def pad_u(n):
        return ((n + _U - 1) // _U) * _U

    chain_lens = tuple([pad_u(b0_slots)] + [pad_u(x) for x in gslots])

    chains = [[[b0_pad[m * 16 + l] if m < b0_slots else (_N_SEG, _N_SPAT)
                for l in range(16)] for m in range(chain_lens[0])]]
    for g in range(_N_GROUPS):
        rows = []
        for m in range(chain_lens[1 + g]):
            row = []
            for l in range(16):
                s = groups[g][l]
                if s is not None and m < counts[s]:
                    row.append(members[s][m])
                else:
                    row.append((_N_SEG, _N_SPAT))
            rows.append(row)
        chains.append(rows)
    pack = np.array([[p[0] + (p[1] << 7) for p in row]
                     for c in chains for row in c], np.int32)
    return chain_lens, res0_lane, pack, np.array(perms, np.int32)


_CHAIN_LENS, _RES0_LANE, _PACK_TAB, _PERM_TAB = _build_tables()
_CHAIN_OFF = tuple(int(x) for x in np.cumsum((0,) + _CHAIN_LENS))
_N_SLOTS = int(sum(_CHAIN_LENS))


def _sc_body(seg_hbm, spat_hbm, pack_hbm, perm_hbm, out_hbm,
             seg_raw, spat_raw, seg_v, spat_v, pack_v, perm_v, out_v):
    wid = lax.axis_index("s") * 2 + lax.axis_index("c")
    base = wid * _ROWS
    pltpu.sync_copy(seg_hbm.at[pl.ds(base, _ROWS)], seg_raw)
    pltpu.sync_copy(spat_hbm.at[pl.ds(base, _ROWS)], spat_raw)
    pltpu.sync_copy(pack_hbm, pack_v)
    pltpu.sync_copy(perm_hbm, perm_v)

    one = jnp.float32(1.0)
    zero16 = jnp.zeros((16,), jnp.float32)
    iota = lax.iota(jnp.int32, 16)

    for r in range(_ROWS):
        row_splat = jnp.zeros((16,), jnp.int32) + r
        for k in range(_SEG_W // 16):
            src = iota + (16 * k)
            g = plsc.load_gather(seg_raw, [row_splat, jnp.minimum(src, _N_SEG - 1)])
            seg_v[pl.ds(r * _SEG_W + 16 * k, 16)] = jnp.where(src < _N_SEG, g, zero16)
        for k in range(_SPAT_W // 16):
            src = iota + (16 * k)
            g = plsc.load_gather(spat_raw, [row_splat, jnp.minimum(src, _N_SPAT - 1)])
            spat_v[pl.ds(r * _SPAT_W + 16 * k, 16)] = jnp.where(src < _N_SPAT, g, zero16)

    def row_body(r, carry):
        seg_off = r * _SEG_W
        spat_off = r * _SPAT_W

        def make_chain(c):
            cbase = _CHAIN_OFF[c] * 16

            def mbody(m, t):
                t0, t1, t2 = t
                moff = m * (_U * 16)
                vs = []
                for u in range(_U):
                    pk = pack_v[pl.ds(moff + (cbase + u * 16), 16)]
                    c1 = (pk & 127) + seg_off
                    c2 = lax.shift_right_logical(pk, 7) + spat_off
                    vs.append(plsc.load_gather(seg_v, [c1])
                              * plsc.load_gather(spat_v, [c2]))
                m01 = jnp.maximum(vs[0], vs[1])
                n01 = jnp.minimum(vs[0], vs[1])
                m23 = jnp.maximum(vs[2], vs[3])
                n23 = jnp.minimum(vs[2], vs[3])
                s0 = jnp.maximum(m01, m23)
                x = jnp.minimum(m01, m23)
                y = jnp.maximum(n01, n23)
                s1 = jnp.maximum(x, y)
                s2 = jnp.minimum(x, y)
                c0 = jnp.maximum(t0, s0)
                c1_ = jnp.maximum(jnp.minimum(t0, s0), jnp.maximum(t1, s1))
                c2_ = jnp.maximum(jnp.maximum(t2, s2),
                                  jnp.maximum(jnp.minimum(t1, s0),
                                              jnp.minimum(t0, s1)))
                return (c0, c1_, c2_)

            return lax.fori_loop(0, _CHAIN_LENS[c] // _U, mbody,
                                 (zero16, zero16, zero16))

        ts = [make_chain(c) for c in range(1 + _N_GROUPS)]

        u0, u1, u2 = ts[0]
        acc = one
        for _ in range(3):
            mx = jnp.max(u0)
            acc = acc * (one - mx)
            msk = iota == plsc.all_reduce_ffs(u0 == mx)
            u0 = jnp.where(msk, u1, u0)
            u1 = jnp.where(msk, u2, u1)
            u2 = jnp.where(msk, zero16, u2)
        res0 = one - acc

        row_splat = jnp.zeros((16,), jnp.int32) + r
        for g in range(_N_GROUPS):
            t0, t1, t2 = ts[1 + g]
            res = one - (one - t0) * (one - t1) * (one - t2)
            if g == _N_GROUPS - 1:
                res = jnp.where(iota == _RES0_LANE, res0, res)
            pos = perm_v[pl.ds(g * 16, 16)]
            plsc.store_scatter(out_v, [row_splat, pos], res)
        return carry

    lax.fori_loop(0, _ROWS, row_body, 0)
    pltpu.sync_copy(out_v, out_hbm.at[pl.ds(base, _ROWS)])


@jax.jit
def _run(seg, spat, pack_flat, perm_flat):
    mesh = plsc.VectorSubcoreMesh(core_axis_name="c", subcore_axis_name="s")
    fn = functools.partial(
        pl.kernel,
        mesh=mesh,
        out_type=jax.ShapeDtypeStruct((_B, _N_OUT), jnp.float32),
        compiler_params=pltpu.CompilerParams(
            needs_layout_passes=False,
            skip_device_barrier=True,
            use_tc_tiling_on_sc=True,
        ),
        scratch_types=[
            pltpu.VMEM((_ROWS, _N_SEG), jnp.float32),
            pltpu.VMEM((_ROWS, _N_SPAT), jnp.float32),
            pltpu.VMEM((_ROWS * _SEG_W,), jnp.float32),
            pltpu.VMEM((_ROWS * _SPAT_W,), jnp.float32),
            pltpu.VMEM((_N_SLOTS * 16,), jnp.int32),
            pltpu.VMEM((_N_GROUPS * 16,), jnp.int32),
            pltpu.VMEM((_ROWS, _N_OUT), jnp.float32),
        ],
    )(_sc_body)
    return fn(seg, spat, pack_flat, perm_flat)


def kernel(seg_confidences, spatial_features, rule_weights):
    del rule_weights
    return _run(seg_confidences, spatial_features,
                jnp.asarray(_PACK_TAB).reshape(-1),
                jnp.asarray(_PERM_TAB).reshape(-1))


# speedup vs baseline: 1.1290x; 1.0431x over previous
import functools

import jax
import jax.numpy as jnp
import numpy as np
from jax import lax
from jax.experimental import pallas as pl
from jax.experimental.pallas import tpu as pltpu
from jax.experimental.pallas import tpu_sc as plsc

_N_SEG = 100
_N_SPAT = 50
_N_OUT = 100
_B = 128
_SEG_W = 112
_SPAT_W = 64
_N_GROUPS = 7
_ROWS = 4
_U = 4


def _build_tables():
    members = [[] for _ in range(_N_OUT)]
    for a in range(_N_SEG):
        for b in range(_N_SPAT):
            s = (a * b) // 10
            if s < _N_OUT:
                members[s].append((a, b))
    counts = [len(m) for m in members]

    b0 = members[0]
    b0_slots = (len(b0) + 15) // 16
    b0_pad = b0 + [(_N_SEG, _N_SPAT)] * (b0_slots * 16 - len(b0))

    rem = sorted(range(1, _N_OUT), key=lambda s: -counts[s])
    group_buckets = [rem[g * 14:(g + 1) * 14] for g in range(6)] + [rem[84:99]]
    later_pool = list(rem[84:99])
    perms, groups = [], []
    res0_lane = None
    pool_i = 0
    for g in range(_N_GROUPS):
        lanes = list(group_buckets[g])
        perm = list(lanes)
        while len(perm) < 16:
            if g == _N_GROUPS - 1 and res0_lane is None:
                res0_lane = len(perm)
                perm.append(0)
            else:
                perm.append(later_pool[pool_i % len(later_pool)])
                pool_i += 1
            lanes.append(None)
        groups.append(lanes)
        perms.append(perm)
    gslots = [max(counts[s] for s in gg if s is not None) for gg in groups]

    def pad_u(n):
        return ((n + _U - 1) // _U) * _U

    chain_lens = tuple([pad_u(b0_slots)] + [pad_u(x) for x in gslots])

    chains = [[[b0_pad[m * 16 + l] if m < b0_slots else (_N_SEG, _N_SPAT)
                for l in range(16)] for m in range(chain_lens[0])]]
    for g in range(_N_GROUPS):
        rows = []
        for m in range(chain_lens[1 + g]):
            row = []
            for l in range(16):
                s = groups[g][l]
                if s is not None and m < counts[s]:
                    row.append(members[s][m])
                else:
                    row.append((_N_SEG, _N_SPAT))
            rows.append(row)
        chains.append(rows)
    pack = np.array([[p[0] + (p[1] << 7) for p in row]
                     for c in chains for row in c], np.int32)
    return chain_lens, res0_lane, pack, np.array(perms, np.int32)


_CHAIN_LENS, _RES0_LANE, _PACK_TAB, _PERM_TAB = _build_tables()
_CHAIN_OFF = tuple(int(x) for x in np.cumsum((0,) + _CHAIN_LENS))
_N_SLOTS = int(sum(_CHAIN_LENS))


def _sc_body(seg_hbm, spat_hbm, pack_hbm, perm_hbm, out_hbm,
             seg_raw, spat_raw, seg_v, spat_v, pack_v, perm_v, out_v, sem):
    wid = lax.axis_index("s") * 2 + lax.axis_index("c")
    base = wid * _ROWS
    c_a = pltpu.make_async_copy(seg_hbm.at[pl.ds(base, _ROWS)], seg_raw, sem)
    c_b = pltpu.make_async_copy(spat_hbm.at[pl.ds(base, _ROWS)], spat_raw, sem)
    c_c = pltpu.make_async_copy(pack_hbm, pack_v, sem)
    c_d = pltpu.make_async_copy(perm_hbm, perm_v, sem)
    c_a.start()
    c_b.start()
    c_c.start()
    c_d.start()
    c_a.wait()
    c_b.wait()
    c_c.wait()
    c_d.wait()

    one = jnp.float32(1.0)
    zero16 = jnp.zeros((16,), jnp.float32)
    iota = lax.iota(jnp.int32, 16)

    for r in range(_ROWS):
        row_splat = jnp.zeros((16,), jnp.int32) + r
        for k in range(_SEG_W // 16):
            src = iota + (16 * k)
            g = plsc.load_gather(seg_raw, [row_splat, jnp.minimum(src, _N_SEG - 1)])
            seg_v[pl.ds(r * _SEG_W + 16 * k, 16)] = jnp.where(src < _N_SEG, g, zero16)
        for k in range(_SPAT_W // 16):
            src = iota + (16 * k)
            g = plsc.load_gather(spat_raw, [row_splat, jnp.minimum(src, _N_SPAT - 1)])
            spat_v[pl.ds(r * _SPAT_W + 16 * k, 16)] = jnp.where(src < _N_SPAT, g, zero16)

    def row_body(r, carry):
        seg_off = r * _SEG_W
        spat_off = r * _SPAT_W

        def make_chain(c):
            cbase = _CHAIN_OFF[c] * 16

            def mbody(m, t):
                t0, t1, t2 = t
                moff = m * (_U * 16)
                vs = []
                for u in range(_U):
                    pk = pack_v[pl.ds(moff + (cbase + u * 16), 16)]
                    c1 = (pk & 127) + seg_off
                    c2 = lax.shift_right_logical(pk, 7) + spat_off
                    vs.append(plsc.load_gather(seg_v, [c1])
                              * plsc.load_gather(spat_v, [c2]))
                m01 = jnp.maximum(vs[0], vs[1])
                n01 = jnp.minimum(vs[0], vs[1])
                m23 = jnp.maximum(vs[2], vs[3])
                n23 = jnp.minimum(vs[2], vs[3])
                s0 = jnp.maximum(m01, m23)
                x = jnp.minimum(m01, m23)
                y = jnp.maximum(n01, n23)
                s1 = jnp.maximum(x, y)
                s2 = jnp.minimum(x, y)
                c0 = jnp.maximum(t0, s0)
                c1_ = jnp.maximum(jnp.minimum(t0, s0), jnp.maximum(t1, s1))
                c2_ = jnp.maximum(jnp.maximum(t2, s2),
                                  jnp.maximum(jnp.minimum(t1, s0),
                                              jnp.minimum(t0, s1)))
                return (c0, c1_, c2_)

            return lax.fori_loop(0, _CHAIN_LENS[c] // _U, mbody,
                                 (zero16, zero16, zero16))

        ts = [make_chain(c) for c in range(1 + _N_GROUPS)]

        u0, u1, u2 = ts[0]
        acc = one
        for _ in range(3):
            mx = jnp.max(u0)
            acc = acc * (one - mx)
            msk = iota == plsc.all_reduce_ffs(u0 == mx)
            u0 = jnp.where(msk, u1, u0)
            u1 = jnp.where(msk, u2, u1)
            u2 = jnp.where(msk, zero16, u2)
        res0 = one - acc

        row_splat = jnp.zeros((16,), jnp.int32) + r
        for g in range(_N_GROUPS):
            t0, t1, t2 = ts[1 + g]
            res = one - (one - t0) * (one - t1) * (one - t2)
            if g == _N_GROUPS - 1:
                res = jnp.where(iota == _RES0_LANE, res0, res)
            pos = perm_v[pl.ds(g * 16, 16)]
            plsc.store_scatter(out_v, [row_splat, pos], res)
        return carry

    lax.fori_loop(0, _ROWS, row_body, 0)
    pltpu.sync_copy(out_v, out_hbm.at[pl.ds(base, _ROWS)])


@jax.jit
def _run(seg, spat, pack_flat, perm_flat):
    mesh = plsc.VectorSubcoreMesh(core_axis_name="c", subcore_axis_name="s")
    fn = functools.partial(
        pl.kernel,
        mesh=mesh,
        out_type=jax.ShapeDtypeStruct((_B, _N_OUT), jnp.float32),
        compiler_params=pltpu.CompilerParams(needs_layout_passes=False),
        scratch_types=[
            pltpu.VMEM((_ROWS, _N_SEG), jnp.float32),
            pltpu.VMEM((_ROWS, _N_SPAT), jnp.float32),
            pltpu.VMEM((_ROWS * _SEG_W,), jnp.float32),
            pltpu.VMEM((_ROWS * _SPAT_W,), jnp.float32),
            pltpu.VMEM((_N_SLOTS * 16,), jnp.int32),
            pltpu.VMEM((_N_GROUPS * 16,), jnp.int32),
            pltpu.VMEM((_ROWS, _N_OUT), jnp.float32),
            pltpu.SemaphoreType.DMA,
        ],
    )(_sc_body)
    return fn(seg, spat, pack_flat, perm_flat)


_DEVICE_TABS = None


def kernel(seg_confidences, spatial_features, rule_weights):
    del rule_weights
    global _DEVICE_TABS
    if _DEVICE_TABS is None:
        _DEVICE_TABS = (jnp.asarray(_PACK_TAB.reshape(-1)),
                        jnp.asarray(_PERM_TAB.reshape(-1)))
    return _run(seg_confidences, spatial_features, *_DEVICE_TABS)
